# Initial kernel scaffold; baseline (speedup 1.0000x reference)
import jax
import jax.numpy as jnp
from jax.experimental import pallas as pl


def kernel(x, data, col_indices, row_ids):
    raise NotImplementedError("write your pallas kernel here")


# speedup vs baseline: 3.0120x; 3.0120x over previous
import functools

import jax
import jax.numpy as jnp
from jax import lax
from jax.experimental import pallas as pl
from jax.experimental.pallas import tpu as pltpu
from jax.experimental.pallas import tpu_sc as plsc

N_ROWS = 16384
BATCH = 256
BH = 128
NW = 32
RB = N_ROWS // NW
ACC_R = RB + 8
K = 128
L = 16
SPL = 48


def _scalar(vec):
    lane = lax.broadcasted_iota(jnp.int32, (L,), 0)
    return jnp.sum(jnp.where(lane == 0, vec, 0))


def _body(xa, xb, colsp, rowsp, datap, splits_hbm, out,
          splits_v, cols_v, rows_v, data_v, G, acc, sem_g, sem_s):
    c_idx = lax.axis_index("c")
    s_idx = lax.axis_index("s")
    w = s_idx * 2 + c_idx
    rowbase = w * RB
    lane = lax.broadcasted_iota(jnp.int32, (L,), 0)
    zero16 = jnp.zeros((L,), jnp.float32)

    pltpu.sync_copy(splits_hbm, splits_v)
    ts = _scalar(plsc.load_gather(splits_v, [jnp.full((L,), w, jnp.int32)]))
    te = _scalar(plsc.load_gather(splits_v, [jnp.full((L,), w + 1, jnp.int32)]))
    lb0 = (ts // 8) * 8
    nc = jnp.maximum((te - lb0 + K - 1) // K, 0)

    for b in range(2):
        xsrc = xa if b == 0 else xb

        def _zrow(r, carry):
            for g in range(BH // L):
                acc[r, pl.ds(g * L, L)] = zero16
            return carry
        lax.fori_loop(0, ACC_R, _zrow, 0)

        def _nz(i, carry):
            lb = carry
            d16 = plsc.load_gather(data_v, [jnp.full((L,), i, jnp.int32)])
            r16 = plsc.load_gather(rows_v, [jnp.full((L,), i, jnp.int32)])
            pos = lb + i
            ok = (pos >= ts) & (pos < te)
            lr16 = jnp.where(ok, r16 - rowbase, RB)
            for g in range(BH // L):
                sl = pl.ds(g * L, L)
                plsc.addupdate_scatter(acc, [lr16, g * L + lane],
                                       G[i, sl] * d16)
            return carry

        def _chunk(cidx, carry):
            lb = lb0 + cidx * K
            d1 = pltpu.async_copy(colsp.at[pl.ds(lb, K)], cols_v, sem_s)
            d2 = pltpu.async_copy(rowsp.at[pl.ds(lb, K)], rows_v, sem_s)
            d3 = pltpu.async_copy(datap.at[pl.ds(lb, K)], data_v, sem_s)
            d1.wait()
            d2.wait()
            d3.wait()
            pltpu.async_copy(xsrc.at[cols_v], G, sem_g).wait()
            lax.fori_loop(0, K, _nz, lb)
            return carry

        lax.fori_loop(0, nc, _chunk, 0)

        pltpu.sync_copy(acc.at[pl.ds(0, RB)],
                        out.at[pl.ds(rowbase, RB), pl.ds(b * BH, BH)])


_spmm = functools.partial(
    pl.kernel,
    out_type=jax.ShapeDtypeStruct((N_ROWS, BATCH), jnp.float32),
    mesh=plsc.VectorSubcoreMesh(core_axis_name="c", subcore_axis_name="s"),
    compiler_params=pltpu.CompilerParams(needs_layout_passes=False),
    scratch_types=[
        pltpu.VMEM((SPL,), jnp.int32),
        pltpu.VMEM((K,), jnp.int32),
        pltpu.VMEM((K,), jnp.int32),
        pltpu.VMEM((K,), jnp.float32),
        pltpu.VMEM((K, BH), jnp.float32),
        pltpu.VMEM((ACC_R, BH), jnp.float32),
        pltpu.SemaphoreType.DMA,
        pltpu.SemaphoreType.DMA,
    ],
)(_body)


def kernel(x, data, col_indices, row_ids):
    nnz = data.shape[0]
    x_t = x.T
    xa = x_t[:, :BH]
    xb = x_t[:, BH:]
    npad = (nnz // K + 2) * K
    pad = npad - nnz
    colsp = jnp.concatenate([col_indices, jnp.zeros((pad,), jnp.int32)])
    rowsp = jnp.concatenate([row_ids, jnp.full((pad,), N_ROWS, jnp.int32)])
    datap = jnp.concatenate([data, jnp.zeros((pad,), jnp.float32)])
    bounds = jnp.searchsorted(
        row_ids, jnp.arange(1, NW, dtype=jnp.int32) * RB, side="left"
    ).astype(jnp.int32)
    splits = jnp.zeros((SPL,), jnp.int32).at[1:NW].set(bounds).at[NW].set(nnz)
    out_t = _spmm(xa, xb, colsp, rowsp, datap, splits)
    return out_t.T


# speedup vs baseline: 3.8363x; 1.2736x over previous
import functools

import jax
import jax.numpy as jnp
from jax import lax
from jax.experimental import pallas as pl
from jax.experimental.pallas import tpu as pltpu
from jax.experimental.pallas import tpu_sc as plsc

N_ROWS = 16384
BATCH = 256
NW = 32
NBLK = 64
RB = N_ROWS // NBLK
ACC_R = RB + 8
K = 96
L = 16
SPL = 72
NG = BATCH // L


def _scalar(vec):
    lane = lax.broadcasted_iota(jnp.int32, (L,), 0)
    return jnp.sum(jnp.where(lane == 0, vec, 0))


def _body(xt, colsp, rowsp, datap, splits_hbm, out,
          splits_v, cols2, rows2, data2, G2, acc, sem_g, sem_s):
    c_idx = lax.axis_index("c")
    s_idx = lax.axis_index("s")
    w = s_idx * 2 + c_idx
    lane = lax.broadcasted_iota(jnp.int32, (L,), 0)
    zero16 = jnp.zeros((L,), jnp.float32)

    pltpu.sync_copy(splits_hbm, splits_v)

    for p in range(2):
        blk = p * NW + w
        rowbase = blk * RB
        ts = _scalar(plsc.load_gather(splits_v, [jnp.full((L,), blk, jnp.int32)]))
        te = _scalar(plsc.load_gather(splits_v, [jnp.full((L,), blk + 1, jnp.int32)]))
        lb0 = (ts // 8) * 8
        nc = jnp.maximum((te - lb0 + K - 1) // K, 0)

        def _zrow(r, carry):
            for g in range(NG):
                acc[r, pl.ds(g * L, L)] = zero16
            return carry
        lax.fori_loop(0, ACC_R, _zrow, 0)

        def _smalls_start(ci, bank):
            lb = lb0 + ci * K
            pltpu.async_copy(colsp.at[pl.ds(lb, K)], cols2.at[bank], sem_s)
            pltpu.async_copy(rowsp.at[pl.ds(lb, K)], rows2.at[bank], sem_s)
            pltpu.async_copy(datap.at[pl.ds(lb, K)], data2.at[bank], sem_s)

        def _smalls_wait(bank):
            pltpu.make_async_copy(colsp.at[pl.ds(0, K)], cols2.at[bank], sem_s).wait()
            pltpu.make_async_copy(rowsp.at[pl.ds(0, K)], rows2.at[bank], sem_s).wait()
            pltpu.make_async_copy(datap.at[pl.ds(0, K)], data2.at[bank], sem_s).wait()

        def _gather_start(bank):
            pltpu.async_copy(xt.at[cols2.at[bank]], G2.at[bank], sem_g)

        def _gather_wait(bank):
            pltpu.make_async_copy(xt.at[cols2.at[bank]], G2.at[bank], sem_g).wait()

        @pl.when(nc > 0)
        def _prologue():
            lb = lb0
            pltpu.sync_copy(colsp.at[pl.ds(lb, K)], cols2.at[0])
            pltpu.sync_copy(rowsp.at[pl.ds(lb, K)], rows2.at[0])
            pltpu.sync_copy(datap.at[pl.ds(lb, K)], data2.at[0])
            _gather_start(0)

        @pl.when(nc > 1)
        def _prologue2():
            _smalls_start(1, 1)

        def _nz(i, carry):
            b2, lb = carry
            d16 = plsc.load_gather(data2.at[b2], [jnp.full((L,), i, jnp.int32)])
            r16 = plsc.load_gather(rows2.at[b2], [jnp.full((L,), i, jnp.int32)])
            pos = lb + i
            ok = (pos >= ts) & (pos < te)
            lr16 = jnp.where(ok, r16 - rowbase, RB)
            for g in range(NG):
                sl = pl.ds(g * L, L)
                plsc.addupdate_scatter(acc, [lr16, g * L + lane],
                                       G2[b2, i, sl] * d16)
            return carry

        def _chunk(ci, carry):
            b2 = lax.rem(ci, 2)
            nb = 1 - b2
            _gather_wait(b2)

            @pl.when(ci + 1 < nc)
            def _next_gather():
                _smalls_wait(nb)
                _gather_start(nb)

            lax.fori_loop(0, K, _nz, (b2, lb0 + ci * K))

            @pl.when(ci + 2 < nc)
            def _next_smalls():
                _smalls_start(ci + 2, b2)

            return carry

        lax.fori_loop(0, nc, _chunk, 0)

        pltpu.sync_copy(acc.at[pl.ds(0, RB)], out.at[pl.ds(rowbase, RB)])


_spmm = functools.partial(
    pl.kernel,
    out_type=jax.ShapeDtypeStruct((N_ROWS, BATCH), jnp.float32),
    mesh=plsc.VectorSubcoreMesh(core_axis_name="c", subcore_axis_name="s"),
    compiler_params=pltpu.CompilerParams(needs_layout_passes=False),
    scratch_types=[
        pltpu.VMEM((SPL,), jnp.int32),
        pltpu.VMEM((2, K), jnp.int32),
        pltpu.VMEM((2, K), jnp.int32),
        pltpu.VMEM((2, K), jnp.float32),
        pltpu.VMEM((2, K, BATCH), jnp.float32),
        pltpu.VMEM((ACC_R, BATCH), jnp.float32),
        pltpu.SemaphoreType.DMA,
        pltpu.SemaphoreType.DMA,
    ],
)(_body)


def kernel(x, data, col_indices, row_ids):
    nnz = data.shape[0]
    x_t = x.T
    npad = (nnz // K + 2) * K
    pad = npad - nnz
    colsp = jnp.concatenate([col_indices, jnp.zeros((pad,), jnp.int32)])
    rowsp = jnp.concatenate([row_ids, jnp.full((pad,), N_ROWS, jnp.int32)])
    datap = jnp.concatenate([data, jnp.zeros((pad,), jnp.float32)])
    bounds = jnp.searchsorted(
        row_ids, jnp.arange(1, NBLK, dtype=jnp.int32) * RB, side="left"
    ).astype(jnp.int32)
    splits = (jnp.zeros((SPL,), jnp.int32)
              .at[1:NBLK].set(bounds).at[NBLK].set(nnz))
    out_t = _spmm(x_t, colsp, rowsp, datap, splits)
    return out_t.T


# speedup vs baseline: 10.0792x; 2.6273x over previous
import functools

import jax
import jax.numpy as jnp
from jax import lax
from jax.experimental import pallas as pl
from jax.experimental.pallas import tpu as pltpu
from jax.experimental.pallas import tpu_sc as plsc

N_ROWS = 16384
BATCH = 256
NW = 32
NBLK = 64
RB = N_ROWS // NBLK
ACC_R = RB + 8
K = 96
L = 16
SPL = 72
NG = BATCH // L


def _scalar(vec):
    lane = lax.broadcasted_iota(jnp.int32, (L,), 0)
    return jnp.sum(jnp.where(lane == 0, vec, 0))


def _body(xt, colsp, rowsp, datap, splits_hbm, out,
          splits_v, cols2, rows2, data2, G2, accf, sem_g, sem_s):
    c_idx = lax.axis_index("c")
    s_idx = lax.axis_index("s")
    w = s_idx * 2 + c_idx
    lane = lax.broadcasted_iota(jnp.int32, (L,), 0)
    zero16 = jnp.zeros((L,), jnp.float32)

    pltpu.sync_copy(splits_hbm, splits_v)

    for p in range(2):
        blk = p * NW + w
        rowbase = blk * RB
        ts = _scalar(plsc.load_gather(splits_v, [jnp.full((L,), blk, jnp.int32)]))
        te = _scalar(plsc.load_gather(splits_v, [jnp.full((L,), blk + 1, jnp.int32)]))
        lb0 = (ts // 8) * 8
        nc = jnp.maximum((te - lb0 + K - 1) // K, 0)

        def _zrow(r, carry):
            for g in range(NG):
                accf[pl.ds(r * BATCH + g * L, L)] = zero16
            return carry
        lax.fori_loop(0, ACC_R, _zrow, 0)

        def _smalls_start(ci, bank):
            lb = lb0 + ci * K
            pltpu.async_copy(colsp.at[pl.ds(lb, K)], cols2.at[bank], sem_s)
            pltpu.async_copy(rowsp.at[pl.ds(lb, K)], rows2.at[bank], sem_s)
            pltpu.async_copy(datap.at[pl.ds(lb, K)], data2.at[bank], sem_s)

        def _smalls_wait(bank):
            pltpu.make_async_copy(colsp.at[pl.ds(0, K)], cols2.at[bank], sem_s).wait()
            pltpu.make_async_copy(rowsp.at[pl.ds(0, K)], rows2.at[bank], sem_s).wait()
            pltpu.make_async_copy(datap.at[pl.ds(0, K)], data2.at[bank], sem_s).wait()

        def _gather_start(bank):
            pltpu.async_copy(xt.at[cols2.at[bank]], G2.at[bank], sem_g)

        def _gather_wait(bank):
            pltpu.make_async_copy(xt.at[cols2.at[bank]], G2.at[bank], sem_g).wait()

        @pl.when(nc > 0)
        def _prologue():
            lb = lb0
            pltpu.sync_copy(colsp.at[pl.ds(lb, K)], cols2.at[0])
            pltpu.sync_copy(rowsp.at[pl.ds(lb, K)], rows2.at[0])
            pltpu.sync_copy(datap.at[pl.ds(lb, K)], data2.at[0])
            _gather_start(0)

        @pl.when(nc > 1)
        def _prologue2():
            _smalls_start(1, 1)

        def _chunk(ci, carry):
            b2 = lax.rem(ci, 2)
            nb = 1 - b2
            lb = lb0 + ci * K
            _gather_wait(b2)

            @pl.when(ci + 1 < nc)
            def _next_gather():
                _smalls_wait(nb)
                _gather_start(nb)

            @plsc.parallel_loop(0, K, step=1, unroll=4)
            def _nz(i):
                sel = jnp.full((L,), i, jnp.int32)
                d16 = plsc.load_gather(data2.at[b2], [sel])
                r16 = plsc.load_gather(rows2.at[b2], [sel])
                pos = lb + i
                ok = (pos >= ts) & (pos < te)
                lr16 = jnp.where(ok, r16 - rowbase, RB)
                base16 = lr16 * BATCH + lane
                for g in range(NG):
                    sl = pl.ds(g * L, L)
                    plsc.addupdate_scatter(accf, [base16 + (g * L)],
                                           G2[b2, i, sl] * d16)

            @pl.when(ci + 2 < nc)
            def _next_smalls():
                _smalls_start(ci + 2, b2)

            return carry

        lax.fori_loop(0, nc, _chunk, 0)

        pltpu.sync_copy(accf.at[pl.ds(0, RB * BATCH)],
                        out.at[pl.ds(rowbase * BATCH, RB * BATCH)])


_spmm = functools.partial(
    pl.kernel,
    out_type=jax.ShapeDtypeStruct((N_ROWS * BATCH,), jnp.float32),
    mesh=plsc.VectorSubcoreMesh(core_axis_name="c", subcore_axis_name="s"),
    compiler_params=pltpu.CompilerParams(needs_layout_passes=False),
    scratch_types=[
        pltpu.VMEM((SPL,), jnp.int32),
        pltpu.VMEM((2, K), jnp.int32),
        pltpu.VMEM((2, K), jnp.int32),
        pltpu.VMEM((2, K), jnp.float32),
        pltpu.VMEM((2, K, BATCH), jnp.float32),
        pltpu.VMEM((ACC_R * BATCH,), jnp.float32),
        pltpu.SemaphoreType.DMA,
        pltpu.SemaphoreType.DMA,
    ],
)(_body)


def kernel(x, data, col_indices, row_ids):
    nnz = data.shape[0]
    x_t = x.T
    npad = (nnz // K + 2) * K
    pad = npad - nnz
    colsp = jnp.concatenate([col_indices, jnp.zeros((pad,), jnp.int32)])
    rowsp = jnp.concatenate([row_ids, jnp.full((pad,), N_ROWS, jnp.int32)])
    datap = jnp.concatenate([data, jnp.zeros((pad,), jnp.float32)])
    bounds = jnp.searchsorted(
        row_ids, jnp.arange(1, NBLK, dtype=jnp.int32) * RB, side="left"
    ).astype(jnp.int32)
    splits = (jnp.zeros((SPL,), jnp.int32)
              .at[1:NBLK].set(bounds).at[NBLK].set(nnz))
    out_t = _spmm(x_t, colsp, rowsp, datap, splits).reshape(N_ROWS, BATCH)
    return out_t.T
